# baseline (speedup 1.0000x reference)
import functools

import jax
import jax.numpy as jnp
import numpy as np
from jax import lax
from jax.experimental import pallas as pl
from jax.experimental.pallas import tpu as pltpu
from jax.experimental.pallas import tpu_sc as plsc

_F = 1024
_ROWS = 2048

_MASKS_NP = np.triu(np.ones((_F, _F), dtype=np.bool_), 1)

_NW = 32
_CHUNK = 32


def _tc_fill_body(idx_ref, z_ref, zo_ref):
    idx = idx_ref[0, 0, :]
    col = jax.lax.broadcasted_iota(jnp.int32, (_ROWS, _F), 1)
    mask = col > idx[:, None]
    zo_ref[...] = jnp.where(mask, jnp.zeros_like(z_ref[...]), z_ref[...])


def _tc_fill(idx3, z2, tokens, f, g):
    return pl.pallas_call(
        _tc_fill_body,
        grid=(g,),
        in_specs=[
            pl.BlockSpec((1, 1, _ROWS), lambda i: (i, 0, 0)),
            pl.BlockSpec((_ROWS, f), lambda i: (i, 0)),
        ],
        out_specs=pl.BlockSpec((_ROWS, f), lambda i: (i, 0)),
        out_shape=jax.ShapeDtypeStruct((tokens, f), z2.dtype),
        compiler_params=pltpu.CompilerParams(
            dimension_semantics=("parallel",),
        ),
    )(idx3, z2)


def _make_sc_mask(tokens):
    per_w = tokens // _NW
    n_chunks = per_w // _CHUNK
    mesh = plsc.VectorSubcoreMesh(core_axis_name="c", subcore_axis_name="s")

    @functools.partial(
        pl.kernel,
        mesh=mesh,
        out_type=jax.ShapeDtypeStruct((tokens, _F), jnp.bool_),
        scratch_types=[
            pltpu.VMEM((_CHUNK,), jnp.int32),
            pltpu.VMEM((_CHUNK, _F), jnp.bool_),
            pltpu.SemaphoreType.DMA,
        ],
    )
    def sc_mask(idx_hbm, table_hbm, out_hbm, idx_v, rows_v, sem):
        wid = lax.axis_index("s") * 2 + lax.axis_index("c")
        base = wid * per_w
        for j in range(n_chunks):
            off = base + j * _CHUNK
            pltpu.sync_copy(idx_hbm.at[pl.ds(off, _CHUNK)], idx_v)
            pltpu.async_copy(table_hbm.at[idx_v], rows_v, sem).wait()
            pltpu.sync_copy(rows_v, out_hbm.at[pl.ds(off, _CHUNK)])

    return sc_mask


def kernel(z):
    b, s, f = z.shape
    tokens = b * s
    idx = jax.random.randint(jax.random.key(1), (b, s), 0, f)
    idx_flat = idx.reshape(tokens).astype(jnp.int32)
    g = tokens // _ROWS
    idx3 = idx_flat.reshape(g, 1, _ROWS)
    z2 = z.reshape(tokens, f)
    table = jnp.asarray(_MASKS_NP)
    mask = _make_sc_mask(tokens)(idx_flat, table)
    zm = _tc_fill(idx3, z2, tokens, f, g)
    return zm.reshape(b, s, f), mask.reshape(b, s, f)


# speedup vs baseline: 1.0365x; 1.0365x over previous
import functools

import jax
import jax.numpy as jnp
import numpy as np
from jax import lax
from jax.experimental import pallas as pl
from jax.experimental.pallas import tpu as pltpu
from jax.experimental.pallas import tpu_sc as plsc

_F = 1024
_ROWS = 2048

_MASKS_NP = np.triu(np.ones((_F, _F), dtype=np.bool_), 1)

_NW = 32
_CHUNK = 32


def _tc_fill_body(idx_ref, z_ref, zo_ref):
    idx = idx_ref[0, 0, :]
    col = jax.lax.broadcasted_iota(jnp.int32, (_ROWS, _F), 1)
    mask = col > idx[:, None]
    zo_ref[...] = jnp.where(mask, jnp.zeros_like(z_ref[...]), z_ref[...])


def _tc_fill(idx3, z2, tokens, f, g):
    return pl.pallas_call(
        _tc_fill_body,
        grid=(g,),
        in_specs=[
            pl.BlockSpec((1, 1, _ROWS), lambda i: (i, 0, 0)),
            pl.BlockSpec((_ROWS, f), lambda i: (i, 0)),
        ],
        out_specs=pl.BlockSpec((_ROWS, f), lambda i: (i, 0)),
        out_shape=jax.ShapeDtypeStruct((tokens, f), z2.dtype),
        compiler_params=pltpu.CompilerParams(
            dimension_semantics=("parallel",),
        ),
    )(idx3, z2)


def _make_sc_mask(tokens):
    per_w = tokens // _NW
    n_chunks = per_w // _CHUNK
    mesh = plsc.VectorSubcoreMesh(core_axis_name="c", subcore_axis_name="s")

    @functools.partial(
        pl.kernel,
        mesh=mesh,
        out_type=jax.ShapeDtypeStruct((tokens, _F), jnp.bool_),
        scratch_types=[
            pltpu.VMEM((tokens // _NW,), jnp.int32),
            pltpu.VMEM((_CHUNK, _F), jnp.bool_),
            pltpu.VMEM((_CHUNK, _F), jnp.bool_),
            pltpu.SemaphoreType.DMA,
            pltpu.SemaphoreType.DMA,
            pltpu.SemaphoreType.DMA,
            pltpu.SemaphoreType.DMA,
        ],
    )
    def sc_mask(idx_hbm, table_hbm, out_hbm, idx_v, rows0, rows1,
                gsem0, gsem1, ssem0, ssem1):
        wid = lax.axis_index("s") * 2 + lax.axis_index("c")
        base = wid * per_w
        pltpu.sync_copy(idx_hbm.at[pl.ds(base, per_w)], idx_v)
        bufs = (rows0, rows1)
        gsems = (gsem0, gsem1)
        ssems = (ssem0, ssem1)
        gets = [None, None]
        puts = [None, None]
        gets[0] = pltpu.async_copy(
            table_hbm.at[idx_v.at[pl.ds(0, _CHUNK)]], bufs[0], gsems[0])
        for j in range(n_chunks):
            cur = j & 1
            nxt = (j + 1) & 1
            if j + 1 < n_chunks:
                if puts[nxt] is not None:
                    puts[nxt].wait()
                gets[nxt] = pltpu.async_copy(
                    table_hbm.at[idx_v.at[pl.ds((j + 1) * _CHUNK, _CHUNK)]],
                    bufs[nxt], gsems[nxt])
            gets[cur].wait()
            puts[cur] = pltpu.async_copy(
                bufs[cur], out_hbm.at[pl.ds(base + j * _CHUNK, _CHUNK)],
                ssems[cur])
        puts[0].wait()
        puts[1].wait()

    return sc_mask


def kernel(z):
    b, s, f = z.shape
    tokens = b * s
    idx = jax.random.randint(jax.random.key(1), (b, s), 0, f)
    idx_flat = idx.reshape(tokens).astype(jnp.int32)
    g = tokens // _ROWS
    idx3 = idx_flat.reshape(g, 1, _ROWS)
    z2 = z.reshape(tokens, f)
    table = jnp.asarray(_MASKS_NP)
    mask = _make_sc_mask(tokens)(idx_flat, table)
    zm = _tc_fill(idx3, z2, tokens, f, g)
    return zm.reshape(b, s, f), mask.reshape(b, s, f)


# speedup vs baseline: 2.3961x; 2.3117x over previous
import functools

import jax
import jax.numpy as jnp
import numpy as np
from jax.experimental import pallas as pl
from jax.experimental.pallas import tpu as pltpu

_F = 1024
_ROWS = 2048


@functools.lru_cache(maxsize=None)
def _idx_const(b, s, f):
    with jax.ensure_compile_time_eval():
        idx = jax.random.randint(jax.random.key(1), (b, s), 0, f)
    return np.asarray(idx, dtype=np.int32)


def _mask_fill_body(idx_ref, z_ref, zo_ref, m_ref):
    idx = idx_ref[0, 0, :]
    col = jax.lax.broadcasted_iota(jnp.int32, (_ROWS, _F), 1)
    mask = col > idx[:, None]
    m_ref[...] = mask.astype(jnp.int8)
    zo_ref[...] = jnp.where(mask, jnp.zeros_like(z_ref[...]), z_ref[...])


def kernel(z):
    b, s, f = z.shape
    tokens = b * s
    g = tokens // _ROWS
    idx3 = jnp.asarray(_idx_const(b, s, f).reshape(g, 1, _ROWS))
    z2 = z.reshape(tokens, f)
    zm, mask8 = pl.pallas_call(
        _mask_fill_body,
        grid=(g,),
        in_specs=[
            pl.BlockSpec((1, 1, _ROWS), lambda i: (i, 0, 0)),
            pl.BlockSpec((_ROWS, f), lambda i: (i, 0)),
        ],
        out_specs=[
            pl.BlockSpec((_ROWS, f), lambda i: (i, 0)),
            pl.BlockSpec((_ROWS, f), lambda i: (i, 0)),
        ],
        out_shape=[
            jax.ShapeDtypeStruct((tokens, f), z.dtype),
            jax.ShapeDtypeStruct((tokens, f), jnp.int8),
        ],
        compiler_params=pltpu.CompilerParams(
            dimension_semantics=("parallel",),
        ),
    )(idx3, z2)
    mask = mask8.astype(jnp.bool_)
    return zm.reshape(b, s, f), mask.reshape(b, s, f)
